# Initial kernel scaffold; baseline (speedup 1.0000x reference)
import jax
import jax.numpy as jnp
from jax.experimental import pallas as pl


def kernel(x, edge_index, edge_attr, params):
    raise NotImplementedError("write your pallas kernel here")


# speedup vs baseline: 4.3895x; 4.3895x over previous
import functools

import jax
import jax.numpy as jnp
from jax import lax
from jax.experimental import pallas as pl
from jax.experimental.pallas import tpu as pltpu
from jax.experimental.pallas import tpu_sc as plsc

N_NODES = 10000
N_EDGES = 640000
D = 128

NW = 32
E_PER_W = N_EDGES // NW
CH = 80
NCH = E_PER_W // CH
N_PAD = 10240
N_PER_TILE = N_PAD // 16

BE = 4000
BN = 2000

_LN_EPS = 1e-5


def _ln(h, g, be):
    mu = jnp.mean(h, axis=-1, keepdims=True)
    var = jnp.mean((h - mu) ** 2, axis=-1, keepdims=True)
    return (h - mu) / jnp.sqrt(var + _LN_EPS) * g + be


def _dot(a, b):
    return jnp.dot(a.astype(jnp.bfloat16), b.astype(jnp.bfloat16),
                   preferred_element_type=jnp.float32)


def _mlp3_body(x_ref, w1, b1, w2, b2, w3, b3, g, be, o_ref, *, ln):
    x = x_ref[...]
    h = jnp.maximum(_dot(x, w1[...]) + b1[...], 0.0)
    h = jnp.maximum(_dot(h, w2[...]) + b2[...], 0.0)
    h = _dot(h, w3[...]) + b3[...]
    if ln:
        h = _ln(h, g[...], be[...])
    o_ref[...] = h


def _w_spec(shape):
    return pl.BlockSpec(shape, lambda i: (0,) * len(shape))


def _mlp3(x, W1, b1, W2, b2, W3, b3, g, be, *, ln, block):
    n, din = x.shape
    dout = W3.shape[1]
    grid = (n // block,)
    in_specs = [
        pl.BlockSpec((block, din), lambda i: (i, 0)),
        _w_spec(W1.shape), _w_spec((1, W1.shape[1])),
        _w_spec(W2.shape), _w_spec((1, W2.shape[1])),
        _w_spec(W3.shape), _w_spec((1, dout)),
        _w_spec((1, dout)), _w_spec((1, dout)),
    ]
    return pl.pallas_call(
        functools.partial(_mlp3_body, ln=ln),
        grid=grid,
        in_specs=in_specs,
        out_specs=pl.BlockSpec((block, dout), lambda i: (i, 0)),
        out_shape=jax.ShapeDtypeStruct((n, dout), jnp.float32),
    )(x, W1, b1.reshape(1, -1), W2, b2.reshape(1, -1), W3, b3.reshape(1, -1),
      g.reshape(1, -1), be.reshape(1, -1))


def _proj2_body(x_ref, wa, wb, oa_ref, ob_ref):
    x = x_ref[...]
    oa_ref[...] = _dot(x, wa[...])
    ob_ref[...] = _dot(x, wb[...])


def _proj2(x, Wa, Wb):
    n = x.shape[0]
    grid = (n // BN,)
    spec = pl.BlockSpec((BN, D), lambda i: (i, 0))
    return pl.pallas_call(
        _proj2_body,
        grid=grid,
        in_specs=[spec, _w_spec((D, D)), _w_spec((D, D))],
        out_specs=[spec, spec],
        out_shape=[jax.ShapeDtypeStruct((n, D), jnp.float32)] * 2,
    )(x, Wa, Wb)


def _edge_step_body(el_ref, gs_ref, gr_ref, w1, b1, w2, b2, w3, b3, g, be,
                    o_ref):
    el = el_ref[...]
    h = jnp.maximum(_dot(el, w1[...]) + gs_ref[...] + gr_ref[...] + b1[...],
                    0.0)
    h = jnp.maximum(_dot(h, w2[...]) + b2[...], 0.0)
    h = _dot(h, w3[...]) + b3[...]
    o_ref[...] = _ln(h, g[...], be[...]) + el


def _edge_step(el, gs, gr, W1e, b1, W2, b2, W3, b3, g, be):
    grid = (N_EDGES // BE,)
    spec = pl.BlockSpec((BE, D), lambda i: (i, 0))
    wspec = _w_spec((D, D))
    bspec = _w_spec((1, D))
    return pl.pallas_call(
        _edge_step_body,
        grid=grid,
        in_specs=[spec, spec, spec, wspec, bspec, wspec, bspec, wspec, bspec,
                  bspec, bspec],
        out_specs=spec,
        out_shape=jax.ShapeDtypeStruct((N_EDGES, D), jnp.float32),
    )(el, gs, gr, W1e, b1.reshape(1, -1), W2, b2.reshape(1, -1), W3,
      b3.reshape(1, -1), g.reshape(1, -1), be.reshape(1, -1))


def _node_step_body(nl_ref, p0_ref, p1_ref, wa, wb, b1, w2, b2, w3, b3, g, be,
                    o_ref):
    nl = nl_ref[...]
    agg = p0_ref[...] + p1_ref[...]
    h = jnp.maximum(_dot(nl, wa[...]) + _dot(agg, wb[...]) + b1[...], 0.0)
    h = jnp.maximum(_dot(h, w2[...]) + b2[...], 0.0)
    h = _dot(h, w3[...]) + b3[...]
    o_ref[...] = _ln(h, g[...], be[...]) + nl


def _node_step(nl, p0, p1, Wa, Wb, b1, W2, b2, W3, b3, g, be):
    grid = (N_NODES // BN,)
    spec = pl.BlockSpec((BN, D), lambda i: (i, 0))
    wspec = _w_spec((D, D))
    bspec = _w_spec((1, D))
    return pl.pallas_call(
        _node_step_body,
        grid=grid,
        in_specs=[spec, spec, spec, wspec, wspec, bspec, wspec, bspec, wspec,
                  bspec, bspec, bspec],
        out_specs=spec,
        out_shape=jax.ShapeDtypeStruct((N_NODES, D), jnp.float32),
    )(nl, p0, p1, Wa, Wb, b1.reshape(1, -1), W2, b2.reshape(1, -1), W3,
      b3.reshape(1, -1), g.reshape(1, -1), be.reshape(1, -1))


@functools.lru_cache(maxsize=None)
def _sc_mesh():
    return plsc.VectorSubcoreMesh(core_axis_name="c", subcore_axis_name="s")


def _sc_gather_body(table_hbm, idx_hbm, out_hbm, idx_v, rows_v, gsem):
    wid = lax.axis_index("s") * 2 + lax.axis_index("c")
    base = pl.multiple_of(wid * E_PER_W, 8)
    pltpu.sync_copy(idx_hbm.at[pl.ds(base, E_PER_W)], idx_v)

    def body(j, carry):
        off = pl.multiple_of(j * CH, 8)
        pltpu.async_copy(
            table_hbm.at[idx_v.at[pl.ds(off, CH)]], rows_v, gsem).wait()
        pltpu.sync_copy(rows_v, out_hbm.at[pl.ds(base + off, CH)])
        return carry

    lax.fori_loop(0, NCH, body, 0)


def _sc_gather(table, idx):
    return pl.kernel(
        _sc_gather_body,
        out_type=jax.ShapeDtypeStruct((N_EDGES, D), jnp.float32),
        mesh=_sc_mesh(),
        scratch_types=[
            pltpu.VMEM((E_PER_W,), jnp.int32),
            pltpu.VMEM((CH, D), jnp.float32),
            pltpu.SemaphoreType.DMA,
        ],
    )(table, idx)


def _sc_scatter_body(vals_hbm, idx_hbm, zeros_hbm, out_hbm, idx_v, buf_v,
                     acc_shared):
    cid = lax.axis_index("c")
    sid = lax.axis_index("s")
    wid = sid * 2 + cid

    @pl.when(sid == 0)
    def _():
        pltpu.sync_copy(zeros_hbm, acc_shared)

    plsc.subcore_barrier()
    pltpu.sync_copy(idx_hbm.at[wid], idx_v)
    base = pl.multiple_of(wid * E_PER_W, 8)

    def body(j, carry):
        off = pl.multiple_of(j * CH, 8)
        pltpu.sync_copy(vals_hbm.at[pl.ds(base + off, CH)], buf_v)
        pltpu.sync_copy(buf_v, acc_shared.at[idx_v.at[j]], add=True)
        return carry

    lax.fori_loop(0, NCH, body, 0)
    plsc.subcore_barrier()
    row = pl.multiple_of(sid * N_PER_TILE, 8)
    pltpu.sync_copy(acc_shared.at[pl.ds(row, N_PER_TILE)],
                    out_hbm.at[cid, pl.ds(row, N_PER_TILE), :])


def _sc_scatter(vals, idx3, zeros):
    return pl.kernel(
        _sc_scatter_body,
        out_type=jax.ShapeDtypeStruct((2, N_PAD, D), jnp.float32),
        mesh=_sc_mesh(),
        scratch_types=[
            pltpu.VMEM((NCH, CH), jnp.int32),
            pltpu.VMEM((CH, D), jnp.float32),
            pltpu.VMEM_SHARED((N_PAD, D), jnp.float32),
        ],
    )(vals, idx3, zeros)


def _mlp_args(p):
    (W1, b1), (W2, b2), (W3, b3) = p["layers"]
    if p["ln"] is not None:
        g, be = p["ln"]
    else:
        g = jnp.ones((W3.shape[1],), jnp.float32)
        be = jnp.zeros((W3.shape[1],), jnp.float32)
    return W1, b1, W2, b2, W3, b3, g, be


def kernel(x, edge_index, edge_attr, params):
    senders = edge_index[0]
    receivers = edge_index[1]
    idx_r3 = receivers.reshape(NW, NCH, CH)
    zeros = jnp.zeros((N_PAD, D), jnp.float32)

    W1, b1, W2, b2, W3, b3, g, be = _mlp_args(params["enc_node"])
    node_lat = _mlp3(x, W1, b1, W2, b2, W3, b3, g, be, ln=True, block=BN)

    W1, b1, W2, b2, W3, b3, g, be = _mlp_args(params["enc_edge"])
    ea = jnp.pad(edge_attr, ((0, 0), (0, 4)))
    W1p = jnp.zeros((8, D), jnp.float32).at[:4].set(W1)
    edge_lat = _mlp3(ea, W1p, b1, W2, b2, W3, b3, g, be, ln=True, block=BE)

    for s in range(len(params["blocks"])):
        blk = params["blocks"][s]
        W1, b1, W2, b2, W3, b3, g, be = _mlp_args(blk["edge"])
        W1e, W1s, W1r = W1[0:D], W1[D:2 * D], W1[2 * D:3 * D]
        Ps, Pr = _proj2(node_lat, W1s, W1r)
        gs = _sc_gather(Ps, senders)
        gr = _sc_gather(Pr, receivers)
        new_e = _edge_step(edge_lat, gs, gr, W1e, b1, W2, b2, W3, b3, g, be)

        parts = _sc_scatter(new_e, idx_r3, zeros)

        W1, b1, W2, b2, W3, b3, g, be = _mlp_args(blk["node"])
        Wa, Wb = W1[0:D], W1[D:2 * D]
        node_lat = _node_step(node_lat, parts[0, :N_NODES], parts[1, :N_NODES],
                              Wa, Wb, b1, W2, b2, W3, b3, g, be)
        edge_lat = new_e

    W1, b1, W2, b2, W3, b3, g, be = _mlp_args(params["dec"])
    W3p = jnp.zeros((D, 8), jnp.float32).at[:, :3].set(W3)
    b3p = jnp.zeros((8,), jnp.float32).at[:3].set(b3)
    out = _mlp3(node_lat, W1, b1, W2, b2, W3p, b3p, g[:1].repeat(8),
                be[:1].repeat(8), ln=False, block=BN)
    return out[:, :3]


# speedup vs baseline: 5.1048x; 1.1629x over previous
import functools

import jax
import jax.numpy as jnp
from jax import lax
from jax.experimental import pallas as pl
from jax.experimental.pallas import tpu as pltpu
from jax.experimental.pallas import tpu_sc as plsc

N_NODES = 10000
N_EDGES = 640000
D = 128

NW = 32
E_PER_W = N_EDGES // NW
CH = 80
NCH = E_PER_W // CH
NCH2 = NCH // 2
N_PAD = 10240
N_PER_TILE = N_PAD // 16

BE = 4000
BN = 2000

_LN_EPS = 1e-5


def _ln(h, g, be):
    mu = jnp.mean(h, axis=-1, keepdims=True)
    var = jnp.mean((h - mu) ** 2, axis=-1, keepdims=True)
    return (h - mu) / jnp.sqrt(var + _LN_EPS) * g + be


def _dot(a, b):
    return jnp.dot(a.astype(jnp.bfloat16), b.astype(jnp.bfloat16),
                   preferred_element_type=jnp.float32)


def _mlp3_body(x_ref, w1, b1, w2, b2, w3, b3, g, be, o_ref, *, ln):
    x = x_ref[...]
    h = jnp.maximum(_dot(x, w1[...]) + b1[...], 0.0)
    h = jnp.maximum(_dot(h, w2[...]) + b2[...], 0.0)
    h = _dot(h, w3[...]) + b3[...]
    if ln:
        h = _ln(h, g[...], be[...])
    o_ref[...] = h


def _w_spec(shape):
    return pl.BlockSpec(shape, lambda i: (0,) * len(shape))


def _mlp3(x, W1, b1, W2, b2, W3, b3, g, be, *, ln, block):
    n, din = x.shape
    dout = W3.shape[1]
    grid = (n // block,)
    in_specs = [
        pl.BlockSpec((block, din), lambda i: (i, 0)),
        _w_spec(W1.shape), _w_spec((1, W1.shape[1])),
        _w_spec(W2.shape), _w_spec((1, W2.shape[1])),
        _w_spec(W3.shape), _w_spec((1, dout)),
        _w_spec((1, dout)), _w_spec((1, dout)),
    ]
    return pl.pallas_call(
        functools.partial(_mlp3_body, ln=ln),
        grid=grid,
        in_specs=in_specs,
        out_specs=pl.BlockSpec((block, dout), lambda i: (i, 0)),
        out_shape=jax.ShapeDtypeStruct((n, dout), jnp.float32),
    )(x, W1, b1.reshape(1, -1), W2, b2.reshape(1, -1), W3, b3.reshape(1, -1),
      g.reshape(1, -1), be.reshape(1, -1))


def _proj2_body(x_ref, wa, wb, oa_ref, ob_ref):
    x = x_ref[...]
    oa_ref[...] = _dot(x, wa[...])
    ob_ref[...] = _dot(x, wb[...])


def _proj2(x, Wa, Wb):
    n = x.shape[0]
    grid = (n // BN,)
    spec = pl.BlockSpec((BN, D), lambda i: (i, 0))
    return pl.pallas_call(
        _proj2_body,
        grid=grid,
        in_specs=[spec, _w_spec((D, D)), _w_spec((D, D))],
        out_specs=[spec, spec],
        out_shape=[jax.ShapeDtypeStruct((n, D), jnp.float32)] * 2,
    )(x, Wa, Wb)


def _edge_step_body(el_ref, gs_ref, gr_ref, w1, b1, w2, b2, w3, b3, g, be,
                    o_ref):
    el = el_ref[...]
    h = jnp.maximum(_dot(el, w1[...]) + gs_ref[...] + gr_ref[...] + b1[...],
                    0.0)
    h = jnp.maximum(_dot(h, w2[...]) + b2[...], 0.0)
    h = _dot(h, w3[...]) + b3[...]
    o_ref[...] = _ln(h, g[...], be[...]) + el


def _edge_step(el, gs, gr, W1e, b1, W2, b2, W3, b3, g, be):
    grid = (N_EDGES // BE,)
    spec = pl.BlockSpec((BE, D), lambda i: (i, 0))
    wspec = _w_spec((D, D))
    bspec = _w_spec((1, D))
    return pl.pallas_call(
        _edge_step_body,
        grid=grid,
        in_specs=[spec, spec, spec, wspec, bspec, wspec, bspec, wspec, bspec,
                  bspec, bspec],
        out_specs=spec,
        out_shape=jax.ShapeDtypeStruct((N_EDGES, D), jnp.float32),
    )(el, gs, gr, W1e, b1.reshape(1, -1), W2, b2.reshape(1, -1), W3,
      b3.reshape(1, -1), g.reshape(1, -1), be.reshape(1, -1))


def _node_step_body(nl_ref, p0_ref, p1_ref, wa, wb, b1, w2, b2, w3, b3, g, be,
                    o_ref):
    nl = nl_ref[...]
    agg = p0_ref[...] + p1_ref[...]
    h = jnp.maximum(_dot(nl, wa[...]) + _dot(agg, wb[...]) + b1[...], 0.0)
    h = jnp.maximum(_dot(h, w2[...]) + b2[...], 0.0)
    h = _dot(h, w3[...]) + b3[...]
    o_ref[...] = _ln(h, g[...], be[...]) + nl


def _node_step(nl, p0, p1, Wa, Wb, b1, W2, b2, W3, b3, g, be):
    grid = (N_NODES // BN,)
    spec = pl.BlockSpec((BN, D), lambda i: (i, 0))
    wspec = _w_spec((D, D))
    bspec = _w_spec((1, D))
    return pl.pallas_call(
        _node_step_body,
        grid=grid,
        in_specs=[spec, spec, spec, wspec, wspec, bspec, wspec, bspec, wspec,
                  bspec, bspec, bspec],
        out_specs=spec,
        out_shape=jax.ShapeDtypeStruct((N_NODES, D), jnp.float32),
    )(nl, p0, p1, Wa, Wb, b1.reshape(1, -1), W2, b2.reshape(1, -1), W3,
      b3.reshape(1, -1), g.reshape(1, -1), be.reshape(1, -1))


@functools.lru_cache(maxsize=None)
def _sc_mesh():
    return plsc.VectorSubcoreMesh(core_axis_name="c", subcore_axis_name="s")


def _sc_gather_body(table_hbm, idx_hbm, out_hbm, idx_v, rows_v, gsem, osem):
    wid = lax.axis_index("s") * 2 + lax.axis_index("c")
    base = pl.multiple_of(wid * E_PER_W, 8)
    pltpu.sync_copy(idx_hbm.at[pl.ds(base, E_PER_W)], idx_v)
    pltpu.async_copy(table_hbm.at[idx_v.at[pl.ds(0, CH)]], rows_v.at[0], gsem)

    def body(j, carry):
        jm = lax.rem(j, 2)
        off = pl.multiple_of(j * CH, 8)
        pltpu.make_async_copy(
            table_hbm.at[idx_v.at[pl.ds(off, CH)]], rows_v.at[jm],
            gsem).wait()

        @pl.when(j >= 1)
        def _():
            offp = pl.multiple_of((j - 1) * CH, 8)
            pltpu.make_async_copy(
                rows_v.at[1 - jm], out_hbm.at[pl.ds(base + offp, CH)],
                osem).wait()

        @pl.when(j + 1 < NCH)
        def _():
            off2 = pl.multiple_of((j + 1) * CH, 8)
            pltpu.async_copy(
                table_hbm.at[idx_v.at[pl.ds(off2, CH)]], rows_v.at[1 - jm],
                gsem)

        pltpu.async_copy(rows_v.at[jm], out_hbm.at[pl.ds(base + off, CH)],
                         osem)
        return carry

    lax.fori_loop(0, NCH, body, 0)
    last = pl.multiple_of((NCH - 1) * CH, 8)
    pltpu.make_async_copy(
        rows_v.at[lax.rem(NCH - 1, 2)], out_hbm.at[pl.ds(base + last, CH)],
        osem).wait()


def _sc_gather(table, idx):
    return pl.kernel(
        _sc_gather_body,
        out_type=jax.ShapeDtypeStruct((N_EDGES, D), jnp.float32),
        mesh=_sc_mesh(),
        scratch_types=[
            pltpu.VMEM((E_PER_W,), jnp.int32),
            pltpu.VMEM((2, CH, D), jnp.float32),
            pltpu.SemaphoreType.DMA,
            pltpu.SemaphoreType.DMA,
        ],
    )(table, idx)


def _sc_scatter_body(vals_hbm, idx_hbm, zeros_hbm, out_hbm, idx_v, buf_v,
                     acc_shared, vsem):
    cid = lax.axis_index("c")
    sid = lax.axis_index("s")
    wid = sid * 2 + cid

    @pl.when(sid == 0)
    def _():
        pltpu.sync_copy(zeros_hbm, acc_shared)

    plsc.subcore_barrier()
    base = pl.multiple_of(wid * E_PER_W, 8)
    for h in range(2):
        pltpu.sync_copy(idx_hbm.at[wid * 2 + h], idx_v)
        hbase = pl.multiple_of(base + h * NCH2 * CH, 8)
        pltpu.async_copy(vals_hbm.at[pl.ds(hbase, CH)], buf_v.at[0], vsem)

        def body(j, carry):
            jm = lax.rem(j, 2)
            off = pl.multiple_of(j * CH, 8)
            pltpu.make_async_copy(vals_hbm.at[pl.ds(hbase + off, CH)],
                                  buf_v.at[jm], vsem).wait()

            @pl.when(j + 1 < NCH2)
            def _():
                off2 = pl.multiple_of((j + 1) * CH, 8)
                pltpu.async_copy(vals_hbm.at[pl.ds(hbase + off2, CH)],
                                 buf_v.at[1 - jm], vsem)

            pltpu.sync_copy(buf_v.at[jm], acc_shared.at[idx_v.at[j]],
                            add=True)
            return carry

        lax.fori_loop(0, NCH2, body, 0)
    plsc.subcore_barrier()
    row = pl.multiple_of(sid * N_PER_TILE, 8)
    pltpu.sync_copy(acc_shared.at[pl.ds(row, N_PER_TILE)],
                    out_hbm.at[cid, pl.ds(row, N_PER_TILE), :])


def _sc_scatter(vals, idx3, zeros):
    return pl.kernel(
        _sc_scatter_body,
        out_type=jax.ShapeDtypeStruct((2, N_PAD, D), jnp.float32),
        mesh=_sc_mesh(),
        scratch_types=[
            pltpu.VMEM((NCH2, CH), jnp.int32),
            pltpu.VMEM((2, CH, D), jnp.float32),
            pltpu.VMEM_SHARED((N_PAD, D), jnp.float32),
            pltpu.SemaphoreType.DMA,
        ],
    )(vals, idx3, zeros)


def _mlp_args(p):
    (W1, b1), (W2, b2), (W3, b3) = p["layers"]
    if p["ln"] is not None:
        g, be = p["ln"]
    else:
        g = jnp.ones((W3.shape[1],), jnp.float32)
        be = jnp.zeros((W3.shape[1],), jnp.float32)
    return W1, b1, W2, b2, W3, b3, g, be


def kernel(x, edge_index, edge_attr, params):
    senders = edge_index[0]
    receivers = edge_index[1]
    idx_r3 = receivers.reshape(NW * 2, NCH2, CH)
    zeros = jnp.zeros((N_PAD, D), jnp.float32)

    W1, b1, W2, b2, W3, b3, g, be = _mlp_args(params["enc_node"])
    node_lat = _mlp3(x, W1, b1, W2, b2, W3, b3, g, be, ln=True, block=BN)

    W1, b1, W2, b2, W3, b3, g, be = _mlp_args(params["enc_edge"])
    ea = jnp.pad(edge_attr, ((0, 0), (0, 4)))
    W1p = jnp.zeros((8, D), jnp.float32).at[:4].set(W1)
    edge_lat = _mlp3(ea, W1p, b1, W2, b2, W3, b3, g, be, ln=True, block=BE)

    for s in range(len(params["blocks"])):
        blk = params["blocks"][s]
        W1, b1, W2, b2, W3, b3, g, be = _mlp_args(blk["edge"])
        W1e, W1s, W1r = W1[0:D], W1[D:2 * D], W1[2 * D:3 * D]
        Ps, Pr = _proj2(node_lat, W1s, W1r)
        gs = _sc_gather(Ps, senders)
        gr = _sc_gather(Pr, receivers)
        new_e = _edge_step(edge_lat, gs, gr, W1e, b1, W2, b2, W3, b3, g, be)

        parts = _sc_scatter(new_e, idx_r3, zeros)

        W1, b1, W2, b2, W3, b3, g, be = _mlp_args(blk["node"])
        Wa, Wb = W1[0:D], W1[D:2 * D]
        node_lat = _node_step(node_lat, parts[0, :N_NODES], parts[1, :N_NODES],
                              Wa, Wb, b1, W2, b2, W3, b3, g, be)
        edge_lat = new_e

    W1, b1, W2, b2, W3, b3, g, be = _mlp_args(params["dec"])
    W3p = jnp.zeros((D, 8), jnp.float32).at[:, :3].set(W3)
    b3p = jnp.zeros((8,), jnp.float32).at[:3].set(b3)
    out = _mlp3(node_lat, W1, b1, W2, b2, W3p, b3p, g[:1].repeat(8),
                be[:1].repeat(8), ln=False, block=BN)
    return out[:, :3]


# speedup vs baseline: 5.3697x; 1.0519x over previous
import functools

import jax
import jax.numpy as jnp
from jax import lax
from jax.experimental import pallas as pl
from jax.experimental.pallas import tpu as pltpu
from jax.experimental.pallas import tpu_sc as plsc

N_NODES = 10000
N_EDGES = 640000
D = 128

NW = 32
E_PER_W = N_EDGES // NW
CH = 80
NCH = E_PER_W // CH
NCH2 = NCH // 2
N_PAD = 10240
N_PER_TILE = N_PAD // 16

BE = 4000
BN = 2000

_LN_EPS = 1e-5


def _ln(h, g, be):
    mu = jnp.mean(h, axis=-1, keepdims=True)
    var = jnp.mean((h - mu) ** 2, axis=-1, keepdims=True)
    return (h - mu) / jnp.sqrt(var + _LN_EPS) * g + be


def _dot(a, b):
    return jnp.dot(a.astype(jnp.bfloat16), b.astype(jnp.bfloat16),
                   preferred_element_type=jnp.float32)


def _mlp3_body(x_ref, w1, b1, w2, b2, w3, b3, g, be, o_ref, *, ln):
    x = x_ref[...]
    h = jnp.maximum(_dot(x, w1[...]) + b1[...], 0.0)
    h = jnp.maximum(_dot(h, w2[...]) + b2[...], 0.0)
    h = _dot(h, w3[...]) + b3[...]
    if ln:
        h = _ln(h, g[...], be[...])
    o_ref[...] = h


def _w_spec(shape):
    return pl.BlockSpec(shape, lambda i: (0,) * len(shape))


def _mlp3(x, W1, b1, W2, b2, W3, b3, g, be, *, ln, block):
    n, din = x.shape
    dout = W3.shape[1]
    grid = (n // block,)
    in_specs = [
        pl.BlockSpec((block, din), lambda i: (i, 0)),
        _w_spec(W1.shape), _w_spec((1, W1.shape[1])),
        _w_spec(W2.shape), _w_spec((1, W2.shape[1])),
        _w_spec(W3.shape), _w_spec((1, dout)),
        _w_spec((1, dout)), _w_spec((1, dout)),
    ]
    return pl.pallas_call(
        functools.partial(_mlp3_body, ln=ln),
        grid=grid,
        in_specs=in_specs,
        out_specs=pl.BlockSpec((block, dout), lambda i: (i, 0)),
        out_shape=jax.ShapeDtypeStruct((n, dout), jnp.float32),
    )(x, W1, b1.reshape(1, -1), W2, b2.reshape(1, -1), W3, b3.reshape(1, -1),
      g.reshape(1, -1), be.reshape(1, -1))


def _proj2_body(x_ref, wa, wb, oa_ref, ob_ref):
    x = x_ref[...]
    oa_ref[...] = _dot(x, wa[...])
    ob_ref[...] = _dot(x, wb[...])


def _proj2(x, Wa, Wb):
    n = x.shape[0]
    grid = (n // BN,)
    spec = pl.BlockSpec((BN, D), lambda i: (i, 0))
    return pl.pallas_call(
        _proj2_body,
        grid=grid,
        in_specs=[spec, _w_spec((D, D)), _w_spec((D, D))],
        out_specs=[spec, spec],
        out_shape=[jax.ShapeDtypeStruct((n, D), jnp.float32)] * 2,
    )(x, Wa, Wb)


def _edge_step_body(el_ref, gs_ref, gr_ref, w1, b1, w2, b2, w3, b3, g, be,
                    o_ref):
    el = el_ref[...]
    h = jnp.maximum(_dot(el, w1[...]) + gs_ref[...] + gr_ref[...] + b1[...],
                    0.0)
    h = jnp.maximum(_dot(h, w2[...]) + b2[...], 0.0)
    h = _dot(h, w3[...]) + b3[...]
    o_ref[...] = _ln(h, g[...], be[...]) + el


def _edge_step(el, gs, gr, W1e, b1, W2, b2, W3, b3, g, be):
    grid = (el.shape[0] // BE,)
    spec = pl.BlockSpec((BE, D), lambda i: (i, 0))
    wspec = _w_spec((D, D))
    bspec = _w_spec((1, D))
    return pl.pallas_call(
        _edge_step_body,
        grid=grid,
        in_specs=[spec, spec, spec, wspec, bspec, wspec, bspec, wspec, bspec,
                  bspec, bspec],
        out_specs=spec,
        out_shape=jax.ShapeDtypeStruct((el.shape[0], D), jnp.float32),
    )(el, gs, gr, W1e, b1.reshape(1, -1), W2, b2.reshape(1, -1), W3,
      b3.reshape(1, -1), g.reshape(1, -1), be.reshape(1, -1))


def _node_step_body(nl_ref, p0_ref, p1_ref, wa, wb, b1, w2, b2, w3, b3, g, be,
                    o_ref):
    nl = nl_ref[...]
    agg = p0_ref[...] + p1_ref[...]
    h = jnp.maximum(_dot(nl, wa[...]) + _dot(agg, wb[...]) + b1[...], 0.0)
    h = jnp.maximum(_dot(h, w2[...]) + b2[...], 0.0)
    h = _dot(h, w3[...]) + b3[...]
    o_ref[...] = _ln(h, g[...], be[...]) + nl


def _node_step(nl, p0, p1, Wa, Wb, b1, W2, b2, W3, b3, g, be):
    grid = (N_NODES // BN,)
    spec = pl.BlockSpec((BN, D), lambda i: (i, 0))
    wspec = _w_spec((D, D))
    bspec = _w_spec((1, D))
    return pl.pallas_call(
        _node_step_body,
        grid=grid,
        in_specs=[spec, spec, spec, wspec, wspec, bspec, wspec, bspec, wspec,
                  bspec, bspec, bspec],
        out_specs=spec,
        out_shape=jax.ShapeDtypeStruct((N_NODES, D), jnp.float32),
    )(nl, p0, p1, Wa, Wb, b1.reshape(1, -1), W2, b2.reshape(1, -1), W3,
      b3.reshape(1, -1), g.reshape(1, -1), be.reshape(1, -1))


@functools.lru_cache(maxsize=None)
def _sc_mesh():
    return plsc.VectorSubcoreMesh(core_axis_name="c", subcore_axis_name="s")


def _make_sc_gather_body(epw, nch):
    def body_fn(table_hbm, idx_hbm, out_hbm, idx_v, rows_v, gsem, osem):
        wid = lax.axis_index("s") * 2 + lax.axis_index("c")
        base = pl.multiple_of(wid * epw, 8)
        pltpu.sync_copy(idx_hbm.at[pl.ds(base, epw)], idx_v)
        pltpu.async_copy(table_hbm.at[idx_v.at[pl.ds(0, CH)]], rows_v.at[0],
                         gsem)

        def body(j, carry):
            jm = lax.rem(j, 2)
            off = pl.multiple_of(j * CH, 8)
            pltpu.make_async_copy(
                table_hbm.at[idx_v.at[pl.ds(off, CH)]], rows_v.at[jm],
                gsem).wait()

            @pl.when(j >= 1)
            def _():
                offp = pl.multiple_of((j - 1) * CH, 8)
                pltpu.make_async_copy(
                    rows_v.at[1 - jm], out_hbm.at[pl.ds(base + offp, CH)],
                    osem).wait()

            @pl.when(j + 1 < nch)
            def _():
                off2 = pl.multiple_of((j + 1) * CH, 8)
                pltpu.async_copy(
                    table_hbm.at[idx_v.at[pl.ds(off2, CH)]], rows_v.at[1 - jm],
                    gsem)

            pltpu.async_copy(rows_v.at[jm], out_hbm.at[pl.ds(base + off, CH)],
                             osem)
            return carry

        lax.fori_loop(0, nch, body, 0)
        last = pl.multiple_of((nch - 1) * CH, 8)
        pltpu.make_async_copy(
            rows_v.at[lax.rem(nch - 1, 2)], out_hbm.at[pl.ds(base + last, CH)],
            osem).wait()

    return body_fn


def _sc_gather(table, idx):
    n_edges = idx.shape[0]
    epw = n_edges // NW
    nch = epw // CH
    return pl.kernel(
        _make_sc_gather_body(epw, nch),
        out_type=jax.ShapeDtypeStruct((n_edges, D), jnp.float32),
        mesh=_sc_mesh(),
        scratch_types=[
            pltpu.VMEM((epw,), jnp.int32),
            pltpu.VMEM((2, CH, D), jnp.float32),
            pltpu.SemaphoreType.DMA,
            pltpu.SemaphoreType.DMA,
        ],
    )(table, idx)


def _sc_scatter_body(v0_hbm, v1_hbm, idx_hbm, zeros_hbm, out_hbm, idx_v,
                     buf_v, acc_shared, vsem):
    cid = lax.axis_index("c")
    sid = lax.axis_index("s")
    wid = sid * 2 + cid

    @pl.when(sid == 0)
    def _():
        pltpu.sync_copy(zeros_hbm, acc_shared)

    plsc.subcore_barrier()

    def accum(vals_hbm, wbase):
        base = pl.multiple_of(wbase * E_PER_W, 8)
        for h in range(2):
            pltpu.sync_copy(idx_hbm.at[wid * 2 + h], idx_v)
            hbase = pl.multiple_of(base + h * NCH2 * CH, 8)
            pltpu.async_copy(vals_hbm.at[pl.ds(hbase, CH)], buf_v.at[0], vsem)

            def body(j, carry):
                jm = lax.rem(j, 2)
                off = pl.multiple_of(j * CH, 8)
                pltpu.make_async_copy(vals_hbm.at[pl.ds(hbase + off, CH)],
                                      buf_v.at[jm], vsem).wait()

                @pl.when(j + 1 < NCH2)
                def _():
                    off2 = pl.multiple_of((j + 1) * CH, 8)
                    pltpu.async_copy(vals_hbm.at[pl.ds(hbase + off2, CH)],
                                     buf_v.at[1 - jm], vsem)

                pltpu.sync_copy(buf_v.at[jm], acc_shared.at[idx_v.at[j]],
                                add=True)
                return carry

            lax.fori_loop(0, NCH2, body, 0)

    @pl.when(wid < NW // 2)
    def _():
        accum(v0_hbm, wid)

    @pl.when(wid >= NW // 2)
    def _():
        accum(v1_hbm, wid - NW // 2)

    plsc.subcore_barrier()
    row = pl.multiple_of(sid * N_PER_TILE, 8)
    pltpu.sync_copy(acc_shared.at[pl.ds(row, N_PER_TILE)],
                    out_hbm.at[cid, pl.ds(row, N_PER_TILE), :])


def _sc_scatter(vals0, vals1, idx3, zeros):
    return pl.kernel(
        _sc_scatter_body,
        out_type=jax.ShapeDtypeStruct((2, N_PAD, D), jnp.float32),
        mesh=_sc_mesh(),
        scratch_types=[
            pltpu.VMEM((NCH2, CH), jnp.int32),
            pltpu.VMEM((2, CH, D), jnp.float32),
            pltpu.VMEM_SHARED((N_PAD, D), jnp.float32),
            pltpu.SemaphoreType.DMA,
        ],
    )(vals0, vals1, idx3, zeros)


def _mlp_args(p):
    (W1, b1), (W2, b2), (W3, b3) = p["layers"]
    if p["ln"] is not None:
        g, be = p["ln"]
    else:
        g = jnp.ones((W3.shape[1],), jnp.float32)
        be = jnp.zeros((W3.shape[1],), jnp.float32)
    return W1, b1, W2, b2, W3, b3, g, be


def kernel(x, edge_index, edge_attr, params):
    senders = edge_index[0]
    receivers = edge_index[1]
    idx_r3 = receivers.reshape(NW * 2, NCH2, CH)
    zeros = jnp.zeros((N_PAD, D), jnp.float32)
    E2 = N_EDGES // 2
    s0, s1 = senders[:E2], senders[E2:]
    r0, r1 = receivers[:E2], receivers[E2:]

    W1, b1, W2, b2, W3, b3, g, be = _mlp_args(params["enc_node"])
    node_lat = _mlp3(x, W1, b1, W2, b2, W3, b3, g, be, ln=True, block=BN)

    W1, b1, W2, b2, W3, b3, g, be = _mlp_args(params["enc_edge"])
    ea = jnp.pad(edge_attr, ((0, 0), (0, 4)))
    W1p = jnp.zeros((8, D), jnp.float32).at[:4].set(W1)
    el0 = _mlp3(ea[:E2], W1p, b1, W2, b2, W3, b3, g, be, ln=True, block=BE)
    el1 = _mlp3(ea[E2:], W1p, b1, W2, b2, W3, b3, g, be, ln=True, block=BE)

    for s in range(len(params["blocks"])):
        blk = params["blocks"][s]
        W1, b1, W2, b2, W3, b3, g, be = _mlp_args(blk["edge"])
        W1e, W1s, W1r = W1[0:D], W1[D:2 * D], W1[2 * D:3 * D]
        Ps, Pr = _proj2(node_lat, W1s, W1r)
        gs0 = _sc_gather(Ps, s0)
        gr0 = _sc_gather(Pr, r0)
        ne0 = _edge_step(el0, gs0, gr0, W1e, b1, W2, b2, W3, b3, g, be)
        gs1 = _sc_gather(Ps, s1)
        gr1 = _sc_gather(Pr, r1)
        ne1 = _edge_step(el1, gs1, gr1, W1e, b1, W2, b2, W3, b3, g, be)

        parts = _sc_scatter(ne0, ne1, idx_r3, zeros)

        W1, b1, W2, b2, W3, b3, g, be = _mlp_args(blk["node"])
        Wa, Wb = W1[0:D], W1[D:2 * D]
        node_lat = _node_step(node_lat, parts[0, :N_NODES], parts[1, :N_NODES],
                              Wa, Wb, b1, W2, b2, W3, b3, g, be)
        el0, el1 = ne0, ne1

    W1, b1, W2, b2, W3, b3, g, be = _mlp_args(params["dec"])
    W3p = jnp.zeros((D, 8), jnp.float32).at[:, :3].set(W3)
    b3p = jnp.zeros((8,), jnp.float32).at[:3].set(b3)
    out = _mlp3(node_lat, W1, b1, W2, b2, W3p, b3p, g[:1].repeat(8),
                be[:1].repeat(8), ln=False, block=BN)
    return out[:, :3]
